# Initial kernel scaffold; baseline (speedup 1.0000x reference)
import jax
import jax.numpy as jnp
from jax.experimental import pallas as pl


def kernel(pred_, label, anchors):
    raise NotImplementedError("write your pallas kernel here")


# speedup vs baseline: 2.3408x; 2.3408x over previous
import dataclasses
import functools

import jax
import jax.numpy as jnp
import numpy as np
from jax import lax
from jax.experimental import pallas as pl
from jax.experimental.pallas import tpu as pltpu
from jax.experimental.pallas import tpu_sc as plsc

GRID = 26
S = GRID * GRID
A = 5
B = 1024
SA = S * A
LANE = SA * A
NROW128 = B * LANE // 128
BB = 64
NSTEPS = B // BB

_ch = np.arange(LANE) % 5
_C1 = jnp.asarray(np.where(_ch < 2, 1.25 / 2704.0, 0.0)[None, :], jnp.float32)
_C3 = jnp.asarray(np.where(_ch == 4, 0.5, 0.0)[None, :], jnp.float32)


def _dense_body(pred_ref, c1_ref, c2_ref, c3_ref, obj_ref, out_ref):
    i = pl.program_id(0)
    v = pred_ref[...]
    e = jnp.exp(v)
    r = 1.0 / (e + 1.0)
    u = e - 1.0
    w = 1.0 - r
    p = c1_ref[...] * (r * r) + c2_ref[...]
    elem = (u * u) * p + c3_ref[...] * (w * w)
    s = jnp.sum(elem)
    acc = jnp.where(i == 0, 0.0, out_ref[0, 0]) + s

    @pl.when(i < NSTEPS - 1)
    def _():
        out_ref[0, 0] = acc

    @pl.when(i == NSTEPS - 1)
    def _():
        ot = obj_ref[...]
        co_sum = jnp.sum(ot[0, :])
        cf_sum = jnp.sum(ot[1, :])
        corr_sum = jnp.sum(ot[2, :])
        out_ref[0, 0] = (5.0 / (4.0 * B)) * co_sum + cf_sum / B \
            + (acc - corr_sum) / (B * (SA - 1))


def _sc_objterms(pred16, lab_t, awp, ahp):
    mesh = plsc.VectorSubcoreMesh(core_axis_name="c", subcore_axis_name="s")
    cp = pltpu.CompilerParams()
    if "needs_layout_passes" in pltpu.CompilerParams.__dataclass_fields__:
        cp = dataclasses.replace(cp, needs_layout_passes=False)

    @functools.partial(
        pl.kernel,
        mesh=mesh,
        compiler_params=cp,
        out_type=jax.ShapeDtypeStruct((3, B), jnp.float32),
        scratch_types=[
            pltpu.VMEM((4, 32), jnp.float32),
            pltpu.VMEM((16,), jnp.float32),
            pltpu.VMEM((16,), jnp.float32),
            pltpu.VMEM((16, 128), jnp.float32),
            pltpu.VMEM((16, 128), jnp.float32),
            pltpu.VMEM((3, 32), jnp.float32),
        ],
    )
    def body(pred_hbm, lab_hbm, aw_hbm, ah_hbm, out_hbm,
             lab_v, aw_v, ah_v, buf_a, buf_b, out_v):
        wid = lax.axis_index("s") * 2 + lax.axis_index("c")
        base = wid * 32
        for c in range(4):
            pltpu.sync_copy(lab_hbm.at[c, pl.ds(base, 32)], lab_v.at[c])
        pltpu.sync_copy(aw_hbm, aw_v)
        pltpu.sync_copy(ah_hbm, ah_v)
        iota = lax.iota(jnp.int32, 16)

        for g in range(2):
            sl = pl.ds(g * 16, 16)
            lx = lab_v[0, sl]
            ly = lab_v[1, sl]
            lw = lab_v[2, sl]
            lh = lab_v[3, sl]
            ixf = (lx * float(GRID)).astype(jnp.int32)
            iyf = (ly * float(GRID)).astype(jnp.int32)
            s_obj = ixf * GRID + iyf
            awvec = aw_v[...]
            ahvec = ah_v[...]
            best_a = jnp.zeros((16,), jnp.int32)
            best_d = None
            for a in range(A):
                awa = jnp.sum(jnp.where(iota == a, awvec, 0.0))
                aha = jnp.sum(jnp.where(iota == a, ahvec, 0.0))
                d = (lw - awa) * (lw - awa) + (lh - aha) * (lh - aha)
                if best_d is None:
                    best_d = d
                else:
                    m = d > best_d
                    best_a = jnp.where(m, a, best_a)
                    best_d = jnp.where(m, d, best_d)
            slot = s_obj * A + best_a
            b_vec = base + g * 16 + iota
            eb = b_vec * LANE + slot * A
            r0 = eb >> 7
            r1 = jnp.minimum(r0 + 1, NROW128 - 1)
            off = eb - (r0 << 7)
            pltpu.sync_copy(pred_hbm.at[r0], buf_a)
            pltpu.sync_copy(pred_hbm.at[r1], buf_b)
            pv = []
            for c in range(5):
                oc = off + c
                va = plsc.load_gather(buf_a, [iota, jnp.minimum(oc, 127)])
                vb = plsc.load_gather(buf_b, [iota, jnp.maximum(oc - 128, 0)])
                pv.append(jnp.where(oc < 128, va, vb))
            p0, p1, p2, p3, p4 = pv
            e0 = jnp.exp(p0)
            e1 = jnp.exp(p1)
            e2 = jnp.exp(p2)
            e3 = jnp.exp(p3)
            e4 = jnp.exp(p4)
            r0f = 1.0 / (e0 + 1.0)
            r1f = 1.0 / (e1 + 1.0)
            sig4 = 1.0 - 1.0 / (e4 + 1.0)
            aw_o = plsc.load_gather(aw_v, [best_a])
            ah_o = plsc.load_gather(ah_v, [best_a])
            px = (ixf.astype(jnp.float32) + (1.0 - r0f)) / float(GRID)
            py = (iyf.astype(jnp.float32) + (1.0 - r1f)) / float(GRID)
            pw = aw_o * e2
            ph = ah_o * e3
            co = ((px - lx) * (px - lx) + (py - ly) * (py - ly)
                  + (pw - lw) * (pw - lw) + (ph - lh) * (ph - lh))
            lx0 = jnp.maximum(lx - lw * 0.5, 0.0)
            ly0 = jnp.maximum(ly - lh * 0.5, 0.0)
            lx1 = jnp.minimum(lx + lw * 0.5, 1.0)
            ly1 = jnp.minimum(ly + lh * 0.5, 1.0)
            px0 = jnp.maximum(px - pw * 0.5, 0.0)
            py0 = jnp.maximum(py - ph * 0.5, 0.0)
            px1 = jnp.minimum(px + pw * 0.5, 1.0)
            py1 = jnp.minimum(py + ph * 0.5, 1.0)
            inter = (jnp.maximum(jnp.minimum(lx1, px1) - jnp.maximum(lx0, px0), 0.0)
                     * jnp.maximum(jnp.minimum(ly1, py1) - jnp.maximum(ly0, py0), 0.0))
            iou = inter / (lw * lh + pw * ph - inter)
            cf = (sig4 - iou) * (sig4 - iou)
            u0 = e0 - 1.0
            u1 = e1 - 1.0
            t0 = u0 * r0f
            t1 = u1 * r1f
            uw = aw_o * (e2 - 1.0)
            uh = ah_o * (e3 - 1.0)
            corr = (1.25 * ((t0 * t0 + t1 * t1) * (1.0 / 2704.0)
                            + uw * uw + uh * uh)
                    + 0.5 * (sig4 * sig4))
            out_v[0, sl] = co
            out_v[1, sl] = cf
            out_v[2, sl] = corr
        for k in range(3):
            pltpu.sync_copy(out_v.at[k], out_hbm.at[k, pl.ds(base, 32)])

    return body(pred16, lab_t, awp, ahp)


def kernel(pred_, label, anchors):
    pred2d = pred_.reshape(B, LANE)
    pred16 = pred_.reshape(NROW128, 128)
    lab_t = label.T
    aw = anchors[:, 0]
    ah = anchors[:, 1]
    awp = jnp.concatenate([aw, jnp.zeros((11,), jnp.float32)])
    ahp = jnp.concatenate([ah, jnp.zeros((11,), jnp.float32)])
    pat = jnp.zeros((25,), jnp.float32)
    pat = pat.at[jnp.arange(5) * 5 + 2].set(1.25 * aw * aw)
    pat = pat.at[jnp.arange(5) * 5 + 3].set(1.25 * ah * ah)
    c2 = jnp.tile(pat, S)[None, :]

    obj = _sc_objterms(pred16, lab_t, awp, ahp)

    out = pl.pallas_call(
        _dense_body,
        grid=(NSTEPS,),
        in_specs=[
            pl.BlockSpec((BB, LANE), lambda i: (i, 0)),
            pl.BlockSpec((1, LANE), lambda i: (0, 0)),
            pl.BlockSpec((1, LANE), lambda i: (0, 0)),
            pl.BlockSpec((1, LANE), lambda i: (0, 0)),
            pl.BlockSpec((3, B), lambda i: (0, 0)),
        ],
        out_specs=pl.BlockSpec(
            (1, 1), lambda i: (0, 0), memory_space=pltpu.SMEM),
        out_shape=jax.ShapeDtypeStruct((1, 1), jnp.float32),
        compiler_params=pltpu.CompilerParams(
            dimension_semantics=("arbitrary",)),
    )(pred2d, _C1, c2, _C3, obj)
    return out.reshape(())
